# Initial kernel scaffold; baseline (speedup 1.0000x reference)
import jax
import jax.numpy as jnp
from jax.experimental import pallas as pl


def kernel(x, edge_index, ids, W1, a1, W2, a2, score_w, score_b):
    raise NotImplementedError("write your pallas kernel here")


# speedup vs baseline: 38.8989x; 38.8989x over previous
import functools

import jax
import jax.numpy as jnp
from jax import lax
from jax.experimental import pallas as pl
from jax.experimental.pallas import tpu as pltpu
from jax.experimental.pallas import tpu_sc as plsc

_N = 10000
_NPAD = 10240
_E = 320000
_H = 8
_DM = 128
_ALPHA = 0.2
_NT = 4

_NC = 2
_NS = 16
_NW = _NC * _NS
_EPT = _E // _NW
_G = 80
_NG = _EPT // _G
_RPT = _NPAD // _NS

_f32 = jnp.float32


def _pre_body(h_ref, w_ref, ac_ref, ac2_ref, wh_ref, es_ref, ed_ref):
    wh = jnp.dot(h_ref[...], w_ref[...], preferred_element_type=_f32)
    wh_ref[...] = wh
    es_ref[...] = jnp.dot(wh, ac_ref[...], preferred_element_type=_f32)
    ed_ref[...] = jnp.dot(wh, ac2_ref[...], preferred_element_type=_f32)


_tc_pre = pl.pallas_call(
    _pre_body,
    out_shape=[
        jax.ShapeDtypeStruct((_N, _DM), _f32),
        jax.ShapeDtypeStruct((_N, 16), _f32),
        jax.ShapeDtypeStruct((_N, 16), _f32),
    ],
)


def _readout(accp, denp, k16):
    acc = accp[0, :_N, :] + accp[1, :_N, :]
    den = denp[0, :_N, :] + denp[1, :_N, :]
    rep = jnp.dot(den, k16, preferred_element_type=_f32)
    out = acc / (rep + 1e-9)
    return jnp.where(out > 0, out, jnp.exp(jnp.minimum(out, 0.0)) - 1.0)


def _mid_body(t, h_ref, accp_ref, denp_ref, ids_ref, w_ref, ac_ref, ac2_ref,
              k16_ref, hn_ref, wh_ref, es_ref, ed_ref):
    hn = _readout(accp_ref[...], denp_ref[...], k16_ref[...])
    h_new = jnp.where(ids_ref[...] == t, hn, h_ref[...])
    hn_ref[...] = h_new
    wh = jnp.dot(h_new, w_ref[...], preferred_element_type=_f32)
    wh_ref[...] = wh
    es_ref[...] = jnp.dot(wh, ac_ref[...], preferred_element_type=_f32)
    ed_ref[...] = jnp.dot(wh, ac2_ref[...], preferred_element_type=_f32)


_tc_mid = [
    pl.pallas_call(
        functools.partial(_mid_body, t),
        out_shape=[
            jax.ShapeDtypeStruct((_N, _DM), _f32),
            jax.ShapeDtypeStruct((_N, _DM), _f32),
            jax.ShapeDtypeStruct((_N, 16), _f32),
            jax.ShapeDtypeStruct((_N, 16), _f32),
        ],
    )
    for t in range(_NT - 1)
]


def _fin_body(h_ref, accp_ref, denp_ref, ids_ref, k16_ref, swt_ref, sb_ref,
              s1_ref, s2_ref):
    hn = _readout(accp_ref[...], denp_ref[...], k16_ref[...])
    h4 = jnp.where(ids_ref[...] == (_NT - 1), hn, h_ref[...])
    s = jnp.sum(h4 * swt_ref[...], axis=1, keepdims=True) + sb_ref[...]
    s = jnp.maximum(s, 0.0)
    for t, out_ref in ((_NT - 2, s1_ref), (_NT - 1, s2_ref)):
        z = jnp.where(ids_ref[...] == t, s, -1e9)
        m = jnp.max(z)
        out_ref[...] = z - m - jnp.log(jnp.sum(jnp.exp(z - m)))


_tc_fin = pl.pallas_call(
    _fin_body,
    out_shape=[
        jax.ShapeDtypeStruct((_N, 1), _f32),
        jax.ShapeDtypeStruct((_N, 1), _f32),
    ],
)


def _lane_bcast(v, idx):
    return lax.gather(
        v, idx[:, None],
        lax.GatherDimensionNumbers(offset_dims=(), collapsed_slice_dims=(0,),
                                   start_index_map=(0,)),
        (1,), mode=lax.GatherScatterMode.PROMISE_IN_BOUNDS)


def _edge_body(es2, ed2, wh, src, dst, accp, denp,
               accs, dens, sgrp, dgrp, sbuf, dbuf, rbuf, mbuf, wstage):
    cid = lax.axis_index("c")
    sid = lax.axis_index("s")
    wid = sid * _NC + cid
    z16 = jnp.zeros((16,), _f32)

    def zrow(j, _):
        for c in range(_DM // 16):
            mbuf[j, pl.ds(c * 16, 16)] = z16
        wstage[j, :] = z16
        return 0

    lax.fori_loop(0, _G, zrow, 0)
    row0 = sid * _RPT
    for k in range(_RPT // _G):
        pltpu.sync_copy(mbuf, accs.at[pl.ds(row0 + k * _G, _G)])
        pltpu.sync_copy(wstage, dens.at[pl.ds(row0 + k * _G, _G)])
    plsc.subcore_barrier()

    eb = wid * _EPT
    bidx = [jnp.full((16,), h, jnp.int32) for h in range(_H)]

    def group(g, _):
        off = eb + g * _G
        pltpu.sync_copy(src.at[pl.ds(off, _G)], sgrp)
        pltpu.sync_copy(dst.at[pl.ds(off, _G)], dgrp)
        pltpu.sync_copy(es2.at[sgrp], sbuf)
        pltpu.sync_copy(ed2.at[dgrp], dbuf)
        pltpu.sync_copy(wh.at[sgrp], rbuf)

        def edge(j, _):
            ev = sbuf[j, :] + dbuf[j, :]
            ev = jnp.where(ev > 0, ev, _ALPHA * ev)
            wv = jnp.exp(ev)
            wstage[j, :] = wv
            for h in range(_H):
                wb = _lane_bcast(wv, bidx[h])
                mbuf[j, pl.ds(h * 16, 16)] = wb * rbuf[j, pl.ds(h * 16, 16)]
            return 0

        lax.fori_loop(0, _G, edge, 0)
        pltpu.sync_copy(mbuf, accs.at[dgrp], add=True)
        pltpu.sync_copy(wstage, dens.at[dgrp], add=True)
        return 0

    lax.fori_loop(0, _NG, group, 0)

    plsc.subcore_barrier()
    pltpu.sync_copy(accs.at[pl.ds(row0, _RPT)], accp.at[cid, pl.ds(row0, _RPT)])
    pltpu.sync_copy(dens.at[pl.ds(row0, _RPT)], denp.at[cid, pl.ds(row0, _RPT)])


_sc_edges = functools.partial(
    pl.kernel,
    mesh=plsc.VectorSubcoreMesh(core_axis_name="c", subcore_axis_name="s"),
    compiler_params=pltpu.CompilerParams(use_tc_tiling_on_sc=False),
    out_type=[
        jax.ShapeDtypeStruct((_NC, _NPAD, _DM), _f32),
        jax.ShapeDtypeStruct((_NC, _NPAD, 16), _f32),
    ],
    scratch_types=[
        pltpu.VMEM_SHARED((_NPAD, _DM), _f32),
        pltpu.VMEM_SHARED((_NPAD, 16), _f32),
        pltpu.VMEM((_G,), jnp.int32),
        pltpu.VMEM((_G,), jnp.int32),
        pltpu.VMEM((_G, 16), _f32),
        pltpu.VMEM((_G, 16), _f32),
        pltpu.VMEM((_G, _DM), _f32),
        pltpu.VMEM((_G, _DM), _f32),
        pltpu.VMEM((_G, 16), _f32),
    ],
)(_edge_body)


def kernel(x, edge_index, ids, W1, a1, W2, a2, score_w, score_b):
    src = edge_index[0]
    dst = edge_index[1]
    ids2 = ids.reshape(_N, 1)
    eye = jnp.eye(_H, dtype=_f32)

    def acat(a):
        a0 = (a[0][:, :, None] * eye[:, None, :]).reshape(_DM, _H)
        a1m = (a[1][:, :, None] * eye[:, None, :]).reshape(_DM, _H)
        return jnp.concatenate([a0, a1m], 1), jnp.concatenate([a1m, a0], 1)

    ac1, ac1b = acat(a1)
    ac2, ac2b = acat(a2)
    wmats = [(W1, ac1, ac1b), (W2, ac2, ac2b)]
    k16 = (jnp.arange(16)[:, None] == (jnp.arange(_DM)[None, :] // 16)).astype(_f32)
    swt = score_w.reshape(1, _DM)
    sb = score_b.reshape(1, 1)

    h = x
    wh, es, ed = _tc_pre(x, W1, ac1, ac1b)
    for t in range(_NT):
        accp, denp = _sc_edges(es, ed, wh, src, dst)
        if t < _NT - 1:
            wn, acn, acnb = wmats[(t + 1) % 2]
            h, wh, es, ed = _tc_mid[t](h, accp, denp, ids2, wn, acn, acnb, k16)
        else:
            s1, s2 = _tc_fin(h, accp, denp, ids2, k16, swt, sb)
    return s1.reshape(_N), s2.reshape(_N)


# speedup vs baseline: 75.9853x; 1.9534x over previous
import functools

import jax
import jax.numpy as jnp
from jax import lax
from jax.experimental import pallas as pl
from jax.experimental.pallas import tpu as pltpu
from jax.experimental.pallas import tpu_sc as plsc

_N = 10000
_NPAD = 10240
_E = 320000
_H = 8
_DM = 128
_ALPHA = 0.2
_NT = 4

_NC = 2
_NS = 16
_HN = _NPAD // _NC
_QN = _HN // 2
_EPT = _E // _NS
_SLAB = 2000
_NSLAB = _EPT // _SLAB
_G = 80
_CCAP = 20480
_ACAP = _CCAP + 16
_RPT = _QN // _NS

_f32 = jnp.float32


def _pre_body(h_ref, w_ref, ac_ref, ac2_ref, ids_ref, wh_ref, es_ref, ed_ref,
              bnd_ref):
    wh = jnp.dot(h_ref[...], w_ref[...], preferred_element_type=_f32)
    wh_ref[...] = wh
    es_ref[...] = jnp.dot(wh, ac_ref[...], preferred_element_type=_f32)
    ed_ref[...] = jnp.dot(wh, ac2_ref[...], preferred_element_type=_f32)
    tgrid = lax.broadcasted_iota(jnp.int32, (1, _DM), 1) // 16
    cmp = (ids_ref[...] < tgrid).astype(_f32)
    bnd_ref[...] = jnp.sum(cmp, axis=0, keepdims=True)


_tc_pre = pl.pallas_call(
    _pre_body,
    out_shape=[
        jax.ShapeDtypeStruct((_N, _DM), _f32),
        jax.ShapeDtypeStruct((_N, 16), _f32),
        jax.ShapeDtypeStruct((_N, 16), _f32),
        jax.ShapeDtypeStruct((1, _DM), _f32),
    ],
)


def _readout(acc_full, den_full, k16):
    acc = acc_full[:_N, :]
    den = den_full[:_N, :]
    rep = jnp.dot(den, k16, preferred_element_type=_f32)
    out = acc / (rep + 1e-9)
    return jnp.where(out > 0, out, jnp.exp(jnp.minimum(out, 0.0)) - 1.0)


def _mid_body(t, h_ref, acc_ref, den_ref, ids_ref, w_ref, ac_ref, ac2_ref,
              k16_ref, hn_ref, wh_ref, es_ref, ed_ref):
    hn = _readout(acc_ref[...], den_ref[...], k16_ref[...])
    h_new = jnp.where(ids_ref[...] == t, hn, h_ref[...])
    hn_ref[...] = h_new
    wh = jnp.dot(h_new, w_ref[...], preferred_element_type=_f32)
    wh_ref[...] = wh
    es_ref[...] = jnp.dot(wh, ac_ref[...], preferred_element_type=_f32)
    ed_ref[...] = jnp.dot(wh, ac2_ref[...], preferred_element_type=_f32)


_tc_mid = [
    pl.pallas_call(
        functools.partial(_mid_body, t),
        out_shape=[
            jax.ShapeDtypeStruct((_N, _DM), _f32),
            jax.ShapeDtypeStruct((_N, _DM), _f32),
            jax.ShapeDtypeStruct((_N, 16), _f32),
            jax.ShapeDtypeStruct((_N, 16), _f32),
        ],
    )
    for t in range(_NT - 1)
]


def _fin_body(h_ref, acc_ref, den_ref, ids_ref, k16_ref, swt_ref, sb_ref,
              s1_ref, s2_ref):
    hn = _readout(acc_ref[...], den_ref[...], k16_ref[...])
    h4 = jnp.where(ids_ref[...] == (_NT - 1), hn, h_ref[...])
    s = jnp.sum(h4 * swt_ref[...], axis=1, keepdims=True) + sb_ref[...]
    s = jnp.maximum(s, 0.0)
    for t, out_ref in ((_NT - 2, s1_ref), (_NT - 1, s2_ref)):
        z = jnp.where(ids_ref[...] == t, s, -1e9)
        m = jnp.max(z)
        out_ref[...] = z - m - jnp.log(jnp.sum(jnp.exp(z - m)))


_tc_fin = pl.pallas_call(
    _fin_body,
    out_shape=[
        jax.ShapeDtypeStruct((_N, 1), _f32),
        jax.ShapeDtypeStruct((_N, 1), _f32),
    ],
)


def _lane_bcast(v, idx):
    return lax.gather(
        v, idx[:, None],
        lax.GatherDimensionNumbers(offset_dims=(), collapsed_slice_dims=(0,),
                                   start_index_map=(0,)),
        (1,), mode=lax.GatherScatterMode.PROMISE_IN_BOUNDS)


def _edge_body(t, es2, ed2, wh, src, dst, bnd, oacc, oden,
               accs, dens, bndv, csrc, cdst,
               sl_s0, sl_s1, sl_d0, sl_d1,
               s0, s1, d0, d1, r0, r1, m0, m1, w0, w1, dg0, dg1,
               slsem0, slsem1, gsem0, gsem1, ssem0, ssem1):
    cid = lax.axis_index("c")
    sid = lax.axis_index("s")
    z16 = jnp.zeros((16,), _f32)
    sl_s = (sl_s0, sl_s1)
    sl_d = (sl_d0, sl_d1)
    sb = (s0, s1)
    db = (d0, d1)
    rb = (r0, r1)
    mb = (m0, m1)
    wst = (w0, w1)
    dg = (dg0, dg1)
    slsem = (slsem0, slsem1)
    gsem = (gsem0, gsem1)
    ssem = (ssem0, ssem1)

    def zrow(j, _):
        for c in range(_DM // 16):
            m0[j, pl.ds(c * 16, 16)] = z16
        w0[j, :] = z16
        return 0

    row0 = sid * _RPT

    def zero_acc():
        lax.fori_loop(0, _G, zrow, 0)
        for k in range(_RPT // _G):
            pltpu.sync_copy(m0, accs.at[pl.ds(row0 + k * _G, _G)])
            pltpu.sync_copy(w0, dens.at[pl.ds(row0 + k * _G, _G)])

    zero_acc()

    pltpu.sync_copy(bnd, bndv)
    lo = lax.convert_element_type(bndv[0, pl.ds(t * 16, 16)][0], jnp.int32)
    hi = lax.convert_element_type(bndv[0, pl.ds((t + 1) * 16, 16)][0], jnp.int32)
    b0 = cid * _HN
    b1 = b0 + _QN
    l0v = jnp.full((16,), jnp.maximum(lo, b0), jnp.int32)
    h0v = jnp.full((16,), jnp.minimum(hi, b1), jnp.int32)
    l1v = jnp.full((16,), jnp.maximum(lo, b1), jnp.int32)
    h1v = jnp.full((16,), jnp.minimum(hi, b1 + _QN), jnp.int32)

    eb = sid * _EPT

    def sl_issue(si, k):
        off = eb + si * _SLAB
        pltpu.async_copy(src.at[pl.ds(off, _SLAB)], sl_s[k], slsem[k])
        pltpu.async_copy(dst.at[pl.ds(off, _SLAB)], sl_d[k], slsem[k])

    def sl_wait(k):
        off = eb
        pltpu.make_async_copy(src.at[pl.ds(off, _SLAB)], sl_s[k], slsem[k]).wait()
        pltpu.make_async_copy(dst.at[pl.ds(off, _SLAB)], sl_d[k], slsem[k]).wait()

    sl_issue(0, 0)
    sl_issue(1, 1)

    def spair(i, cc):
        for k in range(2):
            si = 2 * i + k
            sl_wait(k)

            def vec(v, cc):
                c0, c1 = cc
                sv = sl_s[k][pl.ds(v * 16, 16)]
                dv = sl_d[k][pl.ds(v * 16, 16)]
                m0v = (dv >= l0v) & (dv < h0v)
                m1v = (dv >= l1v) & (dv < h1v)
                p0 = plsc.cumsum(m0v.astype(jnp.int32))
                p1 = plsc.cumsum(m1v.astype(jnp.int32))
                i0 = c0 + p0 - 1
                i1 = (_CCAP - 1) - (c1 + p1 - 1)
                idx = jnp.where(m0v, i0, jnp.where(m1v, i1, _ACAP - 1))
                plsc.store_scatter(cdst, [idx], dv)
                plsc.store_scatter(csrc, [idx], sv)
                return (c0 + p0[15], c1 + p1[15])

            cc = lax.fori_loop(0, _SLAB // 16, vec, cc)

            @pl.when(si + 2 < _NSLAB)
            def _():
                sl_issue(si + 2, k)
        return cc

    c0, c1 = lax.fori_loop(0, _NSLAB // 2, spair, (0, 0))

    zsrc = jnp.zeros((16,), jnp.int32)
    iot = lax.iota(jnp.int32, 16)

    dump0 = jnp.full((16,), b0, jnp.int32)
    k0 = c0 // 16
    r0_ = c0 - k0 * 16
    cdst[pl.ds(k0 * 16, 16)] = jnp.where(iot < r0_, cdst[pl.ds(k0 * 16, 16)],
                                         dump0)
    csrc[pl.ds(k0 * 16, 16)] = jnp.where(iot < r0_, csrc[pl.ds(k0 * 16, 16)],
                                         zsrc)
    ngrp0 = jnp.maximum((c0 + 2 * _G - 1) // (2 * _G) * 2, 2)

    def fill0(v, _):
        cdst[pl.ds(v * 16, 16)] = dump0
        csrc[pl.ds(v * 16, 16)] = zsrc
        return 0

    lax.fori_loop(k0 + 1, ngrp0 * (_G // 16), fill0, 0)

    dump1 = jnp.full((16,), b1, jnp.int32)
    ngrp1 = jnp.maximum((c1 + 2 * _G - 1) // (2 * _G) * 2, 2)
    c1pad = ngrp1 * _G
    start1 = _CCAP - c1pad
    e1 = _CCAP - c1
    ke = e1 // 16
    re = e1 - ke * 16
    cdst[pl.ds(ke * 16, 16)] = jnp.where(iot < re, dump1,
                                         cdst[pl.ds(ke * 16, 16)])
    csrc[pl.ds(ke * 16, 16)] = jnp.where(iot < re, zsrc,
                                         csrc[pl.ds(ke * 16, 16)])

    def fill1(v, _):
        cdst[pl.ds(v * 16, 16)] = dump1
        csrc[pl.ds(v * 16, 16)] = zsrc
        return 0

    lax.fori_loop(start1 // 16, ke, fill1, 0)

    plsc.subcore_barrier()

    bidx = [jnp.full((16,), h, jnp.int32) for h in range(_H)]

    def run_pass(goff, ngrp, llo, lhi, basev):
        def g_issue(g, k):
            ss = csrc.at[pl.ds(goff + g * _G, _G)]
            dd = cdst.at[pl.ds(goff + g * _G, _G)]
            pltpu.async_copy(es2.at[ss], sb[k], gsem[k])
            pltpu.async_copy(ed2.at[dd], db[k], gsem[k])
            pltpu.async_copy(wh.at[ss], rb[k], gsem[k])

        def g_wait(k):
            ss = csrc.at[pl.ds(0, _G)]
            pltpu.make_async_copy(es2.at[ss], sb[k], gsem[k]).wait()
            pltpu.make_async_copy(ed2.at[ss], db[k], gsem[k]).wait()
            pltpu.make_async_copy(wh.at[ss], rb[k], gsem[k]).wait()

        def s_issue(k):
            pltpu.async_copy(mb[k], accs.at[dg[k]], ssem[k], add=True)
            pltpu.async_copy(wst[k], dens.at[dg[k]], ssem[k], add=True)

        def s_wait(k):
            pltpu.make_async_copy(mb[k], accs.at[dg[k]], ssem[k]).wait()
            pltpu.make_async_copy(wst[k], dens.at[dg[k]], ssem[k]).wait()

        def compute(g, k):
            gb = goff + g * _G
            for v in range(_G // 16):
                dg[k][pl.ds(v * 16, 16)] = cdst[pl.ds(gb + v * 16, 16)] - basev
            s0_ = g * _G

            def edge(j, _):
                ev = sb[k][j, :] + db[k][j, :]
                ev = jnp.where(ev > 0, ev, _ALPHA * ev)
                wv = jnp.exp(ev)
                s = s0_ + j
                live = jnp.where((s >= llo) & (s < lhi), 1.0, 0.0)
                wv = wv * live
                wst[k][j, :] = wv
                for h in range(_H):
                    wb = _lane_bcast(wv, bidx[h])
                    mb[k][j, pl.ds(h * 16, 16)] = (
                        wb * rb[k][j, pl.ds(h * 16, 16)])
                return 0

            lax.fori_loop(0, _G, edge, 0)

        g_issue(0, 0)
        g_issue(1, 1)
        npairs = ngrp // 2

        def pair(i, _):
            for k in range(2):
                g = 2 * i + k
                g_wait(k)

                @pl.when(i > 0)
                def _():
                    s_wait(k)

                compute(g, k)
                nxt = g + 2

                @pl.when(nxt < ngrp)
                def _():
                    g_issue(nxt, k)

                s_issue(k)
            return 0

        lax.fori_loop(0, npairs, pair, 0)
        s_wait(0)
        s_wait(1)

    def export(basep):
        orow = basep + row0
        pltpu.sync_copy(accs.at[pl.ds(row0, _RPT)], oacc.at[pl.ds(orow, _RPT)])
        pltpu.sync_copy(dens.at[pl.ds(row0, _RPT)], oden.at[pl.ds(orow, _RPT)])

    run_pass(0, ngrp0, 0, c0, jnp.full((16,), b0, jnp.int32))
    plsc.subcore_barrier()
    export(b0)
    zero_acc()
    plsc.subcore_barrier()
    run_pass(start1, ngrp1, c1pad - c1, c1pad,
             jnp.full((16,), b1, jnp.int32))
    plsc.subcore_barrier()
    export(b1)


def _make_sc_edges(t):
    return functools.partial(
        pl.kernel,
        mesh=plsc.VectorSubcoreMesh(core_axis_name="c", subcore_axis_name="s"),
        compiler_params=pltpu.CompilerParams(use_tc_tiling_on_sc=False,
                                             needs_layout_passes=False),
        out_type=[
            jax.ShapeDtypeStruct((_NPAD, _DM), _f32),
            jax.ShapeDtypeStruct((_NPAD, 16), _f32),
        ],
        scratch_types=(
            [
                pltpu.VMEM_SHARED((_QN, _DM), _f32),
                pltpu.VMEM_SHARED((_QN, 16), _f32),
                pltpu.VMEM((1, _DM), _f32),
                pltpu.VMEM((_ACAP,), jnp.int32),
                pltpu.VMEM((_ACAP,), jnp.int32),
            ]
            + [pltpu.VMEM((_SLAB,), jnp.int32)] * 4
            + [pltpu.VMEM((_G, 16), _f32)] * 2
            + [pltpu.VMEM((_G, 16), _f32)] * 2
            + [pltpu.VMEM((_G, _DM), _f32)] * 2
            + [pltpu.VMEM((_G, _DM), _f32)] * 2
            + [pltpu.VMEM((_G, 16), _f32)] * 2
            + [pltpu.VMEM((_G,), jnp.int32)] * 2
            + [pltpu.SemaphoreType.DMA] * 6
        ),
    )(functools.partial(_edge_body, t))


_sc_edges = [_make_sc_edges(t) for t in range(_NT)]


def kernel(x, edge_index, ids, W1, a1, W2, a2, score_w, score_b):
    src = edge_index[0]
    dst = edge_index[1]
    ids2 = ids.reshape(_N, 1)
    eye = jnp.eye(_H, dtype=_f32)

    def acat(a):
        a0 = (a[0][:, :, None] * eye[:, None, :]).reshape(_DM, _H)
        a1m = (a[1][:, :, None] * eye[:, None, :]).reshape(_DM, _H)
        return jnp.concatenate([a0, a1m], 1), jnp.concatenate([a1m, a0], 1)

    ac1, ac1b = acat(a1)
    ac2, ac2b = acat(a2)
    wmats = [(W1, ac1, ac1b), (W2, ac2, ac2b)]
    k16 = (jnp.arange(16)[:, None] == (jnp.arange(_DM)[None, :] // 16)).astype(_f32)
    swt = score_w.reshape(1, _DM)
    sb = score_b.reshape(1, 1)

    h = x
    wh, es, ed, bnd = _tc_pre(x, W1, ac1, ac1b, ids2)
    for t in range(_NT):
        acc, den = _sc_edges[t](es, ed, wh, src, dst, bnd)
        if t < _NT - 1:
            wn, acn, acnb = wmats[(t + 1) % 2]
            h, wh, es, ed = _tc_mid[t](h, acc, den, ids2, wn, acn, acnb, k16)
        else:
            s1, s2 = _tc_fin(h, acc, den, ids2, k16, swt, sb)
    return s1.reshape(_N), s2.reshape(_N)


# speedup vs baseline: 99.3766x; 1.3078x over previous
import functools

import jax
import jax.numpy as jnp
from jax import lax
from jax.experimental import pallas as pl
from jax.experimental.pallas import tpu as pltpu
from jax.experimental.pallas import tpu_sc as plsc

_N = 10000
_NPAD = 10240
_E = 320000
_H = 8
_DM = 128
_ALPHA = 0.2
_NT = 4

_NC = 2
_NS = 16
_HN = _NPAD // _NC
_QN = _HN // 2
_EPT = _E // _NS
_SLAB = 2000
_NSLAB = _EPT // _SLAB
_G = 80
_CCAP = 20480
_ACAP = _CCAP + 16
_RPT = _QN // _NS

_f32 = jnp.float32


def _pre_body(h_ref, w_ref, ac_ref, ac2_ref, ids_ref, wh_ref, es_ref, ed_ref,
              bnd_ref):
    wh = jnp.dot(h_ref[...], w_ref[...], preferred_element_type=_f32)
    wh_ref[...] = wh
    es_ref[...] = jnp.dot(wh, ac_ref[...], preferred_element_type=_f32)
    ed_ref[...] = jnp.dot(wh, ac2_ref[...], preferred_element_type=_f32)
    tgrid = lax.broadcasted_iota(jnp.int32, (1, _DM), 1) // 16
    cmp = (ids_ref[...] < tgrid).astype(_f32)
    bnd_ref[...] = jnp.sum(cmp, axis=0, keepdims=True)


_tc_pre = pl.pallas_call(
    _pre_body,
    out_shape=[
        jax.ShapeDtypeStruct((_N, _DM), _f32),
        jax.ShapeDtypeStruct((_N, 16), _f32),
        jax.ShapeDtypeStruct((_N, 16), _f32),
        jax.ShapeDtypeStruct((1, _DM), _f32),
    ],
)


def _readout(acc_full, den_full, k16):
    acc = acc_full[:_N, :]
    den = den_full[:_N, :]
    rep = jnp.dot(den, k16, preferred_element_type=_f32)
    out = acc / (rep + 1e-9)
    return jnp.where(out > 0, out, jnp.exp(jnp.minimum(out, 0.0)) - 1.0)


def _mid_body(t, h_ref, acc_ref, den_ref, ids_ref, w_ref, ac_ref, ac2_ref,
              k16_ref, hn_ref, wh_ref, es_ref, ed_ref):
    hn = _readout(acc_ref[...], den_ref[...], k16_ref[...])
    h_new = jnp.where(ids_ref[...] == t, hn, h_ref[...])
    hn_ref[...] = h_new
    wh = jnp.dot(h_new, w_ref[...], preferred_element_type=_f32)
    wh_ref[...] = wh
    es_ref[...] = jnp.dot(wh, ac_ref[...], preferred_element_type=_f32)
    ed_ref[...] = jnp.dot(wh, ac2_ref[...], preferred_element_type=_f32)


_tc_mid = [
    pl.pallas_call(
        functools.partial(_mid_body, t),
        out_shape=[
            jax.ShapeDtypeStruct((_N, _DM), _f32),
            jax.ShapeDtypeStruct((_N, _DM), _f32),
            jax.ShapeDtypeStruct((_N, 16), _f32),
            jax.ShapeDtypeStruct((_N, 16), _f32),
        ],
    )
    for t in range(_NT - 1)
]


def _fin_body(h_ref, acc_ref, den_ref, ids_ref, k16_ref, swt_ref, sb_ref,
              s1_ref, s2_ref):
    hn = _readout(acc_ref[...], den_ref[...], k16_ref[...])
    h4 = jnp.where(ids_ref[...] == (_NT - 1), hn, h_ref[...])
    s = jnp.sum(h4 * swt_ref[...], axis=1, keepdims=True) + sb_ref[...]
    s = jnp.maximum(s, 0.0)
    for t, out_ref in ((_NT - 2, s1_ref), (_NT - 1, s2_ref)):
        z = jnp.where(ids_ref[...] == t, s, -1e9)
        m = jnp.max(z)
        out_ref[...] = z - m - jnp.log(jnp.sum(jnp.exp(z - m)))


_tc_fin = pl.pallas_call(
    _fin_body,
    out_shape=[
        jax.ShapeDtypeStruct((_N, 1), _f32),
        jax.ShapeDtypeStruct((_N, 1), _f32),
    ],
)


def _lane_bcast(v, idx):
    return lax.gather(
        v, idx[:, None],
        lax.GatherDimensionNumbers(offset_dims=(), collapsed_slice_dims=(0,),
                                   start_index_map=(0,)),
        (1,), mode=lax.GatherScatterMode.PROMISE_IN_BOUNDS)


def _edge_body(t, es2, ed2, wh, src, dst, bnd, oacc, oden,
               accs, dens, bndv, csrc, cdst,
               sl_s0, sl_s1, sl_d0, sl_d1,
               s0, s1, d0, d1, r0, r1, m0, m1, w0, w1, dg0, dg1,
               slsem0, slsem1, gsem0, gsem1, ssem0, ssem1):
    cid = lax.axis_index("c")
    sid = lax.axis_index("s")
    z16 = jnp.zeros((16,), _f32)
    sl_s = (sl_s0, sl_s1)
    sl_d = (sl_d0, sl_d1)
    sb = (s0, s1)
    db = (d0, d1)
    rb = (r0, r1)
    mb = (m0, m1)
    wst = (w0, w1)
    dg = (dg0, dg1)
    slsem = (slsem0, slsem1)
    gsem = (gsem0, gsem1)
    ssem = (ssem0, ssem1)

    row0 = sid * _RPT

    def zero_acc():
        @plsc.parallel_loop(0, _G, unroll=8)
        def _(j):
            for c in range(_DM // 16):
                m0[j, pl.ds(c * 16, 16)] = z16
            w0[j, :] = z16

        for k in range(_RPT // _G):
            pltpu.sync_copy(m0, accs.at[pl.ds(row0 + k * _G, _G)])
            pltpu.sync_copy(w0, dens.at[pl.ds(row0 + k * _G, _G)])

    zero_acc()

    pltpu.sync_copy(bnd, bndv)
    lo = lax.convert_element_type(bndv[0, pl.ds(t * 16, 16)][0], jnp.int32)
    hi = lax.convert_element_type(bndv[0, pl.ds((t + 1) * 16, 16)][0], jnp.int32)
    b0 = cid * _HN
    b1 = b0 + _QN
    l0v = jnp.full((16,), jnp.maximum(lo, b0), jnp.int32)
    h0v = jnp.full((16,), jnp.minimum(hi, b1), jnp.int32)
    l1v = jnp.full((16,), jnp.maximum(lo, b1), jnp.int32)
    h1v = jnp.full((16,), jnp.minimum(hi, b1 + _QN), jnp.int32)

    eb = sid * _EPT

    def sl_issue(si, k):
        off = eb + si * _SLAB
        pltpu.async_copy(src.at[pl.ds(off, _SLAB)], sl_s[k], slsem[k])
        pltpu.async_copy(dst.at[pl.ds(off, _SLAB)], sl_d[k], slsem[k])

    def sl_wait(k):
        off = eb
        pltpu.make_async_copy(src.at[pl.ds(off, _SLAB)], sl_s[k], slsem[k]).wait()
        pltpu.make_async_copy(dst.at[pl.ds(off, _SLAB)], sl_d[k], slsem[k]).wait()

    sl_issue(0, 0)
    sl_issue(1, 1)

    def spair(i, cc):
        for k in range(2):
            si = 2 * i + k
            sl_wait(k)

            @plsc.parallel_loop(0, _SLAB // 16, unroll=4, carry=cc)
            def vec(v, cc):
                c0, c1 = cc
                sv = sl_s[k][pl.ds(v * 16, 16)]
                dv = sl_d[k][pl.ds(v * 16, 16)]
                m0v = (dv >= l0v) & (dv < h0v)
                m1v = (dv >= l1v) & (dv < h1v)
                p0 = plsc.cumsum(m0v.astype(jnp.int32))
                p1 = plsc.cumsum(m1v.astype(jnp.int32))
                i0 = c0 + p0 - 1
                i1 = (_CCAP - 1) - (c1 + p1 - 1)
                idx = jnp.where(m0v, i0, jnp.where(m1v, i1, _ACAP - 1))
                plsc.store_scatter(cdst, [idx], dv)
                plsc.store_scatter(csrc, [idx], sv)
                return (c0 + p0[15], c1 + p1[15])

            cc = vec

            @pl.when(si + 2 < _NSLAB)
            def _():
                sl_issue(si + 2, k)
        return cc

    c0, c1 = lax.fori_loop(0, _NSLAB // 2, spair, (0, 0))

    zsrc = jnp.zeros((16,), jnp.int32)
    iot = lax.iota(jnp.int32, 16)

    dump0 = jnp.full((16,), b0, jnp.int32)
    k0 = c0 // 16
    r0_ = c0 - k0 * 16
    cdst[pl.ds(k0 * 16, 16)] = jnp.where(iot < r0_, cdst[pl.ds(k0 * 16, 16)],
                                         dump0)
    csrc[pl.ds(k0 * 16, 16)] = jnp.where(iot < r0_, csrc[pl.ds(k0 * 16, 16)],
                                         zsrc)
    ngrp0 = jnp.maximum((c0 + 2 * _G - 1) // (2 * _G) * 2, 2)

    def fill0(v, _):
        cdst[pl.ds(v * 16, 16)] = dump0
        csrc[pl.ds(v * 16, 16)] = zsrc
        return 0

    lax.fori_loop(k0 + 1, ngrp0 * (_G // 16), fill0, 0)

    dump1 = jnp.full((16,), b1, jnp.int32)
    ngrp1 = jnp.maximum((c1 + 2 * _G - 1) // (2 * _G) * 2, 2)
    c1pad = ngrp1 * _G
    start1 = _CCAP - c1pad
    e1 = _CCAP - c1
    ke = e1 // 16
    re = e1 - ke * 16
    cdst[pl.ds(ke * 16, 16)] = jnp.where(iot < re, dump1,
                                         cdst[pl.ds(ke * 16, 16)])
    csrc[pl.ds(ke * 16, 16)] = jnp.where(iot < re, zsrc,
                                         csrc[pl.ds(ke * 16, 16)])

    def fill1(v, _):
        cdst[pl.ds(v * 16, 16)] = dump1
        csrc[pl.ds(v * 16, 16)] = zsrc
        return 0

    lax.fori_loop(start1 // 16, ke, fill1, 0)

    plsc.subcore_barrier()

    bidx = [jnp.full((16,), h, jnp.int32) for h in range(_H)]

    def run_pass(goff, ngrp, llo, lhi, basev):
        def g_issue(g, k):
            ss = csrc.at[pl.ds(goff + g * _G, _G)]
            dd = cdst.at[pl.ds(goff + g * _G, _G)]
            pltpu.async_copy(es2.at[ss], sb[k], gsem[k])
            pltpu.async_copy(ed2.at[dd], db[k], gsem[k])
            pltpu.async_copy(wh.at[ss], rb[k], gsem[k])

        def g_wait(k):
            ss = csrc.at[pl.ds(0, _G)]
            pltpu.make_async_copy(es2.at[ss], sb[k], gsem[k]).wait()
            pltpu.make_async_copy(ed2.at[ss], db[k], gsem[k]).wait()
            pltpu.make_async_copy(wh.at[ss], rb[k], gsem[k]).wait()

        def s_issue(k):
            pltpu.async_copy(mb[k], accs.at[dg[k]], ssem[k], add=True)
            pltpu.async_copy(wst[k], dens.at[dg[k]], ssem[k], add=True)

        def s_wait(k):
            pltpu.make_async_copy(mb[k], accs.at[dg[k]], ssem[k]).wait()
            pltpu.make_async_copy(wst[k], dens.at[dg[k]], ssem[k]).wait()

        def compute(g, k):
            gb = goff + g * _G
            for v in range(_G // 16):
                dg[k][pl.ds(v * 16, 16)] = cdst[pl.ds(gb + v * 16, 16)] - basev
            s0_ = g * _G

            @plsc.parallel_loop(0, _G, unroll=8)
            def edge(j):
                ev = sb[k][j, :] + db[k][j, :]
                ev = jnp.where(ev > 0, ev, _ALPHA * ev)
                wv = jnp.exp(ev)
                s = s0_ + j
                live = jnp.where((s >= llo) & (s < lhi), 1.0, 0.0)
                wv = wv * live
                wst[k][j, :] = wv
                for h in range(_H):
                    wb = _lane_bcast(wv, bidx[h])
                    mb[k][j, pl.ds(h * 16, 16)] = (
                        wb * rb[k][j, pl.ds(h * 16, 16)])

        g_issue(0, 0)
        g_issue(1, 1)
        npairs = ngrp // 2

        def pair(i, _):
            for k in range(2):
                g = 2 * i + k
                g_wait(k)

                @pl.when(i > 0)
                def _():
                    s_wait(k)

                compute(g, k)
                nxt = g + 2

                @pl.when(nxt < ngrp)
                def _():
                    g_issue(nxt, k)

                s_issue(k)
            return 0

        lax.fori_loop(0, npairs, pair, 0)
        s_wait(0)
        s_wait(1)

    def export(basep):
        orow = basep + row0
        pltpu.sync_copy(accs.at[pl.ds(row0, _RPT)], oacc.at[pl.ds(orow, _RPT)])
        pltpu.sync_copy(dens.at[pl.ds(row0, _RPT)], oden.at[pl.ds(orow, _RPT)])

    run_pass(0, ngrp0, 0, c0, jnp.full((16,), b0, jnp.int32))
    plsc.subcore_barrier()
    export(b0)
    zero_acc()
    plsc.subcore_barrier()
    run_pass(start1, ngrp1, c1pad - c1, c1pad,
             jnp.full((16,), b1, jnp.int32))
    plsc.subcore_barrier()
    export(b1)


def _make_sc_edges(t):
    return functools.partial(
        pl.kernel,
        mesh=plsc.VectorSubcoreMesh(core_axis_name="c", subcore_axis_name="s"),
        compiler_params=pltpu.CompilerParams(use_tc_tiling_on_sc=False,
                                             needs_layout_passes=False),
        out_type=[
            jax.ShapeDtypeStruct((_NPAD, _DM), _f32),
            jax.ShapeDtypeStruct((_NPAD, 16), _f32),
        ],
        scratch_types=(
            [
                pltpu.VMEM_SHARED((_QN, _DM), _f32),
                pltpu.VMEM_SHARED((_QN, 16), _f32),
                pltpu.VMEM((1, _DM), _f32),
                pltpu.VMEM((_ACAP,), jnp.int32),
                pltpu.VMEM((_ACAP,), jnp.int32),
            ]
            + [pltpu.VMEM((_SLAB,), jnp.int32)] * 4
            + [pltpu.VMEM((_G, 16), _f32)] * 2
            + [pltpu.VMEM((_G, 16), _f32)] * 2
            + [pltpu.VMEM((_G, _DM), _f32)] * 2
            + [pltpu.VMEM((_G, _DM), _f32)] * 2
            + [pltpu.VMEM((_G, 16), _f32)] * 2
            + [pltpu.VMEM((_G,), jnp.int32)] * 2
            + [pltpu.SemaphoreType.DMA] * 6
        ),
    )(functools.partial(_edge_body, t))


_sc_edges = [_make_sc_edges(t) for t in range(_NT)]


def kernel(x, edge_index, ids, W1, a1, W2, a2, score_w, score_b):
    src = edge_index[0]
    dst = edge_index[1]
    ids2 = ids.reshape(_N, 1)
    eye = jnp.eye(_H, dtype=_f32)

    def acat(a):
        a0 = (a[0][:, :, None] * eye[:, None, :]).reshape(_DM, _H)
        a1m = (a[1][:, :, None] * eye[:, None, :]).reshape(_DM, _H)
        return jnp.concatenate([a0, a1m], 1), jnp.concatenate([a1m, a0], 1)

    ac1, ac1b = acat(a1)
    ac2, ac2b = acat(a2)
    wmats = [(W1, ac1, ac1b), (W2, ac2, ac2b)]
    k16 = (jnp.arange(16)[:, None] == (jnp.arange(_DM)[None, :] // 16)).astype(_f32)
    swt = score_w.reshape(1, _DM)
    sb = score_b.reshape(1, 1)

    h = x
    wh, es, ed, bnd = _tc_pre(x, W1, ac1, ac1b, ids2)
    for t in range(_NT):
        acc, den = _sc_edges[t](es, ed, wh, src, dst, bnd)
        if t < _NT - 1:
            wn, acn, acnb = wmats[(t + 1) % 2]
            h, wh, es, ed = _tc_mid[t](h, acc, den, ids2, wn, acn, acnb, k16)
        else:
            s1, s2 = _tc_fin(h, acc, den, ids2, k16, swt, sb)
    return s1.reshape(_N), s2.reshape(_N)


# speedup vs baseline: 136.9369x; 1.3780x over previous
import functools

import jax
import jax.numpy as jnp
from jax import lax
from jax.experimental import pallas as pl
from jax.experimental.pallas import tpu as pltpu
from jax.experimental.pallas import tpu_sc as plsc

_N = 10000
_NPAD = 10240
_E = 320000
_H = 8
_DM = 128
_ALPHA = 0.2
_NT = 4

_NC = 2
_NS = 16
_HN = _NPAD // _NC
_QN = _HN // 2
_EPT = _E // _NS
_SLAB = 2000
_NSLAB = _EPT // _SLAB
_G = 80
_CCAP = 20480
_ACAP = _CCAP + 16
_RPT = _QN // _NS

_f32 = jnp.float32


def _pre_body(h_ref, w_ref, ac_ref, ac2_ref, ids_ref, wh_ref, es_ref, ed_ref,
              bnd_ref):
    wh = jnp.dot(h_ref[...], w_ref[...], preferred_element_type=_f32)
    wh_ref[...] = wh
    es_ref[...] = jnp.dot(wh, ac_ref[...], preferred_element_type=_f32)
    ed_ref[...] = jnp.dot(wh, ac2_ref[...], preferred_element_type=_f32)
    tgrid = lax.broadcasted_iota(jnp.int32, (1, _DM), 1) // 16
    cmp = (ids_ref[...] < tgrid).astype(_f32)
    bnd_ref[...] = jnp.sum(cmp, axis=0, keepdims=True)


_tc_pre = pl.pallas_call(
    _pre_body,
    out_shape=[
        jax.ShapeDtypeStruct((_N, _DM), _f32),
        jax.ShapeDtypeStruct((_N, 16), _f32),
        jax.ShapeDtypeStruct((_N, 16), _f32),
        jax.ShapeDtypeStruct((1, _DM), _f32),
    ],
)


def _readout(acc_full, den_full, k16):
    acc = acc_full[:_N, :]
    den = den_full[:_N, :]
    rep = jnp.dot(den, k16, preferred_element_type=_f32)
    out = acc / (rep + 1e-9)
    return jnp.where(out > 0, out, jnp.exp(jnp.minimum(out, 0.0)) - 1.0)


def _mid_body(t, h_ref, acc_ref, den_ref, ids_ref, w_ref, ac_ref, ac2_ref,
              k16_ref, hn_ref, wh_ref, es_ref, ed_ref):
    hn = _readout(acc_ref[...], den_ref[...], k16_ref[...])
    h_new = jnp.where(ids_ref[...] == t, hn, h_ref[...])
    hn_ref[...] = h_new
    wh = jnp.dot(h_new, w_ref[...], preferred_element_type=_f32)
    wh_ref[...] = wh
    es_ref[...] = jnp.dot(wh, ac_ref[...], preferred_element_type=_f32)
    ed_ref[...] = jnp.dot(wh, ac2_ref[...], preferred_element_type=_f32)


_tc_mid = [
    pl.pallas_call(
        functools.partial(_mid_body, t),
        out_shape=[
            jax.ShapeDtypeStruct((_N, _DM), _f32),
            jax.ShapeDtypeStruct((_N, _DM), _f32),
            jax.ShapeDtypeStruct((_N, 16), _f32),
            jax.ShapeDtypeStruct((_N, 16), _f32),
        ],
    )
    for t in range(_NT - 1)
]


def _fin_body(h_ref, acc_ref, den_ref, ids_ref, k16_ref, swt_ref, sb_ref,
              s1_ref, s2_ref):
    hn = _readout(acc_ref[...], den_ref[...], k16_ref[...])
    h4 = jnp.where(ids_ref[...] == (_NT - 1), hn, h_ref[...])
    s = jnp.sum(h4 * swt_ref[...], axis=1, keepdims=True) + sb_ref[...]
    s = jnp.maximum(s, 0.0)
    for t, out_ref in ((_NT - 2, s1_ref), (_NT - 1, s2_ref)):
        z = jnp.where(ids_ref[...] == t, s, -1e9)
        m = jnp.max(z)
        out_ref[...] = z - m - jnp.log(jnp.sum(jnp.exp(z - m)))


_tc_fin = pl.pallas_call(
    _fin_body,
    out_shape=[
        jax.ShapeDtypeStruct((_N, 1), _f32),
        jax.ShapeDtypeStruct((_N, 1), _f32),
    ],
)


def _lane_bcast(v, idx):
    return lax.gather(
        v, idx[:, None],
        lax.GatherDimensionNumbers(offset_dims=(), collapsed_slice_dims=(0,),
                                   start_index_map=(0,)),
        (1,), mode=lax.GatherScatterMode.PROMISE_IN_BOUNDS)


def _edge_body(t, es2, ed2, wh, src, dst, bnd, oacc, oden,
               accs, dens, bndv, csrc, cdst,
               sl_s0, sl_s1, sl_d0, sl_d1,
               s0, s1, d0, d1, r0, r1, m0, m1, w0, w1, dg0, dg1,
               slsem0, slsem1, gsem0, gsem1, ssem0, ssem1):
    cid = lax.axis_index("c")
    sid = lax.axis_index("s")
    z16 = jnp.zeros((16,), _f32)
    sl_s = (sl_s0, sl_s1)
    sl_d = (sl_d0, sl_d1)
    sb = (s0, s1)
    db = (d0, d1)
    rb = (r0, r1)
    mb = (m0, m1)
    wst = (w0, w1)
    dg = (dg0, dg1)
    slsem = (slsem0, slsem1)
    gsem = (gsem0, gsem1)
    ssem = (ssem0, ssem1)

    row0 = sid * _RPT

    def zero_acc():
        @plsc.parallel_loop(0, _G, unroll=8)
        def _(j):
            for c in range(_DM // 16):
                m0[j, pl.ds(c * 16, 16)] = z16
            w0[j, :] = z16

        for k in range(_RPT // _G):
            pltpu.sync_copy(m0, accs.at[pl.ds(row0 + k * _G, _G)])
            pltpu.sync_copy(w0, dens.at[pl.ds(row0 + k * _G, _G)])

    zero_acc()

    pltpu.sync_copy(bnd, bndv)
    lo = lax.convert_element_type(bndv[0, pl.ds(t * 16, 16)][0], jnp.int32)
    hi = lax.convert_element_type(bndv[0, pl.ds((t + 1) * 16, 16)][0], jnp.int32)
    seg = hi - lo
    q = ((seg + 4 * _RPT - 1) // (4 * _RPT)) * _RPT
    qb0 = lo + cid * q
    qb1 = lo + (cid + 2) * q
    l0v = jnp.full((16,), qb0, jnp.int32)
    h0v = jnp.full((16,), jnp.minimum(qb0 + q, hi), jnp.int32)
    l1v = jnp.full((16,), qb1, jnp.int32)
    h1v = jnp.full((16,), jnp.minimum(qb1 + q, hi), jnp.int32)

    eb = sid * _EPT

    def sl_issue(si, k):
        off = eb + si * _SLAB
        pltpu.async_copy(src.at[pl.ds(off, _SLAB)], sl_s[k], slsem[k])
        pltpu.async_copy(dst.at[pl.ds(off, _SLAB)], sl_d[k], slsem[k])

    def sl_wait(k):
        off = eb
        pltpu.make_async_copy(src.at[pl.ds(off, _SLAB)], sl_s[k], slsem[k]).wait()
        pltpu.make_async_copy(dst.at[pl.ds(off, _SLAB)], sl_d[k], slsem[k]).wait()

    sl_issue(0, 0)
    sl_issue(1, 1)

    def spair(i, cc):
        for k in range(2):
            si = 2 * i + k
            sl_wait(k)

            @plsc.parallel_loop(0, _SLAB // 16, unroll=4, carry=cc)
            def vec(v, cc):
                c0, c1 = cc
                sv = sl_s[k][pl.ds(v * 16, 16)]
                dv = sl_d[k][pl.ds(v * 16, 16)]
                m0v = (dv >= l0v) & (dv < h0v)
                m1v = (dv >= l1v) & (dv < h1v)
                p0 = plsc.cumsum(m0v.astype(jnp.int32))
                p1 = plsc.cumsum(m1v.astype(jnp.int32))
                i0 = c0 + p0 - 1
                i1 = (_CCAP - 1) - (c1 + p1 - 1)
                idx = jnp.where(m0v, i0, jnp.where(m1v, i1, _ACAP - 1))
                plsc.store_scatter(cdst, [idx], dv)
                plsc.store_scatter(csrc, [idx], sv)
                return (c0 + p0[15], c1 + p1[15])

            cc = vec

            @pl.when(si + 2 < _NSLAB)
            def _():
                sl_issue(si + 2, k)
        return cc

    c0, c1 = lax.fori_loop(0, _NSLAB // 2, spair, (0, 0))

    zsrc = jnp.zeros((16,), jnp.int32)
    iot = lax.iota(jnp.int32, 16)

    dump0 = zsrc
    k0 = c0 // 16
    r0_ = c0 - k0 * 16
    cdst[pl.ds(k0 * 16, 16)] = jnp.where(iot < r0_, cdst[pl.ds(k0 * 16, 16)],
                                         dump0)
    csrc[pl.ds(k0 * 16, 16)] = jnp.where(iot < r0_, csrc[pl.ds(k0 * 16, 16)],
                                         zsrc)
    ngrp0 = jnp.maximum((c0 + 2 * _G - 1) // (2 * _G) * 2, 2)

    def fill0(v, _):
        cdst[pl.ds(v * 16, 16)] = dump0
        csrc[pl.ds(v * 16, 16)] = zsrc
        return 0

    lax.fori_loop(k0 + 1, ngrp0 * (_G // 16), fill0, 0)

    dump1 = zsrc
    ngrp1 = jnp.maximum((c1 + 2 * _G - 1) // (2 * _G) * 2, 2)
    c1pad = ngrp1 * _G
    start1 = _CCAP - c1pad
    e1 = _CCAP - c1
    ke = e1 // 16
    re = e1 - ke * 16
    cdst[pl.ds(ke * 16, 16)] = jnp.where(iot < re, dump1,
                                         cdst[pl.ds(ke * 16, 16)])
    csrc[pl.ds(ke * 16, 16)] = jnp.where(iot < re, zsrc,
                                         csrc[pl.ds(ke * 16, 16)])

    def fill1(v, _):
        cdst[pl.ds(v * 16, 16)] = dump1
        csrc[pl.ds(v * 16, 16)] = zsrc
        return 0

    lax.fori_loop(start1 // 16, ke, fill1, 0)

    plsc.subcore_barrier()

    bidx = [jnp.full((16,), h, jnp.int32) for h in range(_H)]

    def run_pass(goff, ngrp, llo, lhi, basev):
        def g_issue(g, k):
            ss = csrc.at[pl.ds(goff + g * _G, _G)]
            dd = cdst.at[pl.ds(goff + g * _G, _G)]
            pltpu.async_copy(es2.at[ss], sb[k], gsem[k])
            pltpu.async_copy(ed2.at[dd], db[k], gsem[k])
            pltpu.async_copy(wh.at[ss], rb[k], gsem[k])

        def g_wait(k):
            ss = csrc.at[pl.ds(0, _G)]
            pltpu.make_async_copy(es2.at[ss], sb[k], gsem[k]).wait()
            pltpu.make_async_copy(ed2.at[ss], db[k], gsem[k]).wait()
            pltpu.make_async_copy(wh.at[ss], rb[k], gsem[k]).wait()

        def s_issue(k):
            pltpu.async_copy(mb[k], accs.at[dg[k]], ssem[k], add=True)
            pltpu.async_copy(wst[k], dens.at[dg[k]], ssem[k], add=True)

        def s_wait(k):
            pltpu.make_async_copy(mb[k], accs.at[dg[k]], ssem[k]).wait()
            pltpu.make_async_copy(wst[k], dens.at[dg[k]], ssem[k]).wait()

        def compute(g, k):
            gb = goff + g * _G
            for v in range(_G // 16):
                dg[k][pl.ds(v * 16, 16)] = jnp.maximum(
                    cdst[pl.ds(gb + v * 16, 16)] - basev, 0)
            s0_ = g * _G

            @plsc.parallel_loop(0, _G, unroll=8)
            def edge(j):
                ev = sb[k][j, :] + db[k][j, :]
                ev = jnp.where(ev > 0, ev, _ALPHA * ev)
                wv = jnp.exp(ev)
                s = s0_ + j
                live = jnp.where((s >= llo) & (s < lhi), 1.0, 0.0)
                wv = wv * live
                wst[k][j, :] = wv
                for h in range(_H):
                    wb = _lane_bcast(wv, bidx[h])
                    mb[k][j, pl.ds(h * 16, 16)] = (
                        wb * rb[k][j, pl.ds(h * 16, 16)])

        g_issue(0, 0)
        g_issue(1, 1)
        npairs = ngrp // 2

        def pair(i, _):
            for k in range(2):
                g = 2 * i + k
                g_wait(k)

                @pl.when(i > 0)
                def _():
                    s_wait(k)

                compute(g, k)
                nxt = g + 2

                @pl.when(nxt < ngrp)
                def _():
                    g_issue(nxt, k)

                s_issue(k)
            return 0

        lax.fori_loop(0, npairs, pair, 0)
        s_wait(0)
        s_wait(1)

    def export(qb):
        @pl.when(row0 < q)
        def _():
            orow = qb + row0
            pltpu.sync_copy(accs.at[pl.ds(row0, _RPT)],
                            oacc.at[pl.ds(orow, _RPT)])
            pltpu.sync_copy(dens.at[pl.ds(row0, _RPT)],
                            oden.at[pl.ds(orow, _RPT)])

    run_pass(0, ngrp0, 0, c0, jnp.full((16,), qb0, jnp.int32))
    plsc.subcore_barrier()
    export(qb0)
    zero_acc()
    plsc.subcore_barrier()
    run_pass(start1, ngrp1, c1pad - c1, c1pad,
             jnp.full((16,), qb1, jnp.int32))
    plsc.subcore_barrier()
    export(qb1)


def _make_sc_edges(t):
    return functools.partial(
        pl.kernel,
        mesh=plsc.VectorSubcoreMesh(core_axis_name="c", subcore_axis_name="s"),
        compiler_params=pltpu.CompilerParams(use_tc_tiling_on_sc=False,
                                             needs_layout_passes=False),
        out_type=[
            jax.ShapeDtypeStruct((_NPAD, _DM), _f32),
            jax.ShapeDtypeStruct((_NPAD, 16), _f32),
        ],
        scratch_types=(
            [
                pltpu.VMEM_SHARED((_QN, _DM), _f32),
                pltpu.VMEM_SHARED((_QN, 16), _f32),
                pltpu.VMEM((1, _DM), _f32),
                pltpu.VMEM((_ACAP,), jnp.int32),
                pltpu.VMEM((_ACAP,), jnp.int32),
            ]
            + [pltpu.VMEM((_SLAB,), jnp.int32)] * 4
            + [pltpu.VMEM((_G, 16), _f32)] * 2
            + [pltpu.VMEM((_G, 16), _f32)] * 2
            + [pltpu.VMEM((_G, _DM), _f32)] * 2
            + [pltpu.VMEM((_G, _DM), _f32)] * 2
            + [pltpu.VMEM((_G, 16), _f32)] * 2
            + [pltpu.VMEM((_G,), jnp.int32)] * 2
            + [pltpu.SemaphoreType.DMA] * 6
        ),
    )(functools.partial(_edge_body, t))


_sc_edges = [_make_sc_edges(t) for t in range(_NT)]


def kernel(x, edge_index, ids, W1, a1, W2, a2, score_w, score_b):
    src = edge_index[0]
    dst = edge_index[1]
    ids2 = ids.reshape(_N, 1)
    eye = jnp.eye(_H, dtype=_f32)

    def acat(a):
        a0 = (a[0][:, :, None] * eye[:, None, :]).reshape(_DM, _H)
        a1m = (a[1][:, :, None] * eye[:, None, :]).reshape(_DM, _H)
        return jnp.concatenate([a0, a1m], 1), jnp.concatenate([a1m, a0], 1)

    ac1, ac1b = acat(a1)
    ac2, ac2b = acat(a2)
    wmats = [(W1, ac1, ac1b), (W2, ac2, ac2b)]
    k16 = (jnp.arange(16)[:, None] == (jnp.arange(_DM)[None, :] // 16)).astype(_f32)
    swt = score_w.reshape(1, _DM)
    sb = score_b.reshape(1, 1)

    h = x
    wh, es, ed, bnd = _tc_pre(x, W1, ac1, ac1b, ids2)
    for t in range(_NT):
        acc, den = _sc_edges[t](es, ed, wh, src, dst, bnd)
        if t < _NT - 1:
            wn, acn, acnb = wmats[(t + 1) % 2]
            h, wh, es, ed = _tc_mid[t](h, acc, den, ids2, wn, acn, acnb, k16)
        else:
            s1, s2 = _tc_fin(h, acc, den, ids2, k16, swt, sb)
    return s1.reshape(_N), s2.reshape(_N)
